# baseline (speedup 1.0000x reference)
import functools

import jax
import jax.numpy as jnp
from jax import lax
from jax.experimental import pallas as pl
from jax.experimental.pallas import tpu as pltpu
from jax.experimental.pallas import tpu_sc as plsc

ROWS = 4096
COLS = 4096
LANES = 16
NCHUNK = COLS // LANES

NUM_CORES = 2
NUM_SUBCORES = 16
NUM_WORKERS = NUM_CORES * NUM_SUBCORES

SC_ROWS = 1536
RPW = SC_ROWS // NUM_WORKERS
SC_ROW0 = ROWS - SC_ROWS

TC_ROWS = ROWS - SC_ROWS
ROW_BLOCK = 128
NUM_BLOCKS = TC_ROWS // ROW_BLOCK
NBUF = 4
CHUNK_LO = TC_ROWS // NUM_WORKERS


def _argmax_body(x_hbm, idx_ref, buf, sem):
    i = pl.program_id(0)
    slot = lax.rem(i, NBUF)

    @pl.when(i == 0)
    def _():
        for j in range(NBUF - 1):
            pltpu.make_async_copy(
                x_hbm.at[pl.ds(j * ROW_BLOCK, ROW_BLOCK), 0],
                buf.at[j],
                sem.at[j],
            ).start()

    @pl.when(i + NBUF - 1 < NUM_BLOCKS)
    def _():
        nxt = lax.rem(i + NBUF - 1, NBUF)
        pltpu.make_async_copy(
            x_hbm.at[pl.ds((i + NBUF - 1) * ROW_BLOCK, ROW_BLOCK), 0],
            buf.at[nxt],
            sem.at[nxt],
        ).start()

    pltpu.make_async_copy(
        x_hbm.at[pl.ds(i * ROW_BLOCK, ROW_BLOCK), 0], buf.at[slot], sem.at[slot]
    ).wait()

    x = buf[slot]
    m = jnp.max(x, axis=1, keepdims=True)
    col = lax.broadcasted_iota(jnp.int32, x.shape, 1)
    cand = jnp.where(x == m, col, COLS)
    idx_ref[0, 0, :] = jnp.min(cand, axis=1)


_argmax_call = pl.pallas_call(
    _argmax_body,
    grid=(NUM_BLOCKS,),
    in_specs=[pl.BlockSpec(memory_space=pltpu.MemorySpace.HBM)],
    out_specs=pl.BlockSpec((1, 1, ROW_BLOCK), lambda i: (i, 0, 0)),
    out_shape=jax.ShapeDtypeStruct((NUM_BLOCKS, 1, ROW_BLOCK), jnp.int32),
    scratch_shapes=[
        pltpu.VMEM((NBUF, ROW_BLOCK, COLS), jnp.float32),
        pltpu.SemaphoreType.DMA((NBUF,)),
    ],
)

_SC_MESH = plsc.VectorSubcoreMesh(
    core_axis_name="c", subcore_axis_name="s", num_cores=NUM_CORES
)


def _row_argmax(rb):
    def chunk_body(c, carry):
        m, mi, cidx = carry
        v = rb[pl.ds(c * LANES, LANES)]
        pred = v > m
        m = jnp.where(pred, v, m)
        mi = jnp.where(pred, cidx, mi)
        return m, mi, cidx + LANES

    m0 = jnp.full((LANES,), -jnp.inf, jnp.float32)
    i0 = jnp.zeros((LANES,), jnp.int32)
    c0 = lax.iota(jnp.int32, LANES)
    m, mi, _ = lax.fori_loop(0, NCHUNK, chunk_body, (m0, i0, c0), unroll=8)
    gm = m
    for s in (1, 2, 4, 8):
        gm = jnp.maximum(gm, _rotate(gm, s))
    cand = jnp.where(m == gm, mi, COLS)
    for s in (1, 2, 4, 8):
        cand = jnp.minimum(cand, _rotate(cand, s))
    return cand


def _rotate(x, s):
    perm = ((lax.iota(jnp.int32, LANES) + s) & (LANES - 1)).reshape(LANES, 1)
    dnums = lax.GatherDimensionNumbers(
        offset_dims=(), collapsed_slice_dims=(0,), start_index_map=(0,)
    )
    return lax.gather(
        x, perm, dnums, (1,), mode=lax.GatherScatterMode.PROMISE_IN_BOUNDS
    )


@functools.partial(
    pl.kernel,
    out_type=jax.ShapeDtypeStruct((SC_ROWS,), jnp.float32),
    scratch_types=[
        pltpu.VMEM((2, COLS), jnp.float32),
        pltpu.VMEM((RPW,), jnp.int32),
        pltpu.VMEM((RPW,), jnp.float32),
        pltpu.SemaphoreType.DMA((2,)),
        pltpu.SemaphoreType.DMA,
    ],
    mesh=_SC_MESH,
)
def _sc_argmax_lookup(x_hbm, fsp_hbm, out_hbm, row_buf, idx_v, out_v, sems, gsem):
    wid = lax.axis_index("s") * NUM_CORES + lax.axis_index("c")
    r0 = SC_ROW0 + wid * RPW

    pltpu.make_async_copy(x_hbm.at[r0, 0], row_buf.at[0], sems.at[0]).start()
    kiota = lax.iota(jnp.int32, LANES)

    def group_body(g, _):
        acc = jnp.zeros((LANES,), jnp.int32)
        base = g * LANES
        for k in range(LANES):
            rr = base + k
            slot = k % 2
            nslot = 1 - slot

            @pl.when(rr + 1 < RPW)
            def _():
                pltpu.make_async_copy(
                    x_hbm.at[r0 + rr + 1, 0], row_buf.at[nslot], sems.at[nslot]
                ).start()

            pltpu.make_async_copy(
                x_hbm.at[r0 + rr, 0], row_buf.at[slot], sems.at[slot]
            ).wait()
            giv = _row_argmax(row_buf.at[slot])
            acc = jnp.where(kiota == k, giv, acc)
        idx_v[pl.ds(base, LANES)] = acc
        return 0

    lax.fori_loop(0, RPW // LANES, group_body, 0)
    pltpu.async_copy(fsp_hbm.at[idx_v], out_v, gsem).wait()
    pltpu.sync_copy(out_v, out_hbm.at[pl.ds(wid * RPW, RPW)])


@functools.partial(
    pl.kernel,
    out_type=jax.ShapeDtypeStruct((TC_ROWS,), jnp.float32),
    scratch_types=[
        pltpu.VMEM((CHUNK_LO,), jnp.int32),
        pltpu.VMEM((CHUNK_LO,), jnp.float32),
        pltpu.SemaphoreType.DMA,
    ],
    mesh=_SC_MESH,
)
def _sc_gather(idx_hbm, fsp_hbm, out_hbm, idx_v, out_v, sem):
    wid = lax.axis_index("s") * NUM_CORES + lax.axis_index("c")
    base = wid * CHUNK_LO
    pltpu.sync_copy(idx_hbm.at[pl.ds(base, CHUNK_LO)], idx_v)
    pltpu.async_copy(fsp_hbm.at[idx_v], out_v, sem).wait()
    pltpu.sync_copy(out_v, out_hbm.at[pl.ds(base, CHUNK_LO)])


def kernel(input, fspace):
    out_hi = _sc_argmax_lookup(input, fspace)
    idx_lo = _argmax_call(input).reshape(TC_ROWS)
    out_lo = _sc_gather(idx_lo, fspace)
    return jnp.concatenate([out_lo, out_hi])


# speedup vs baseline: 1.3284x; 1.3284x over previous
import functools

import jax
import jax.numpy as jnp
from jax import lax
from jax.experimental import pallas as pl
from jax.experimental.pallas import tpu as pltpu
from jax.experimental.pallas import tpu_sc as plsc

ROWS = 4096
COLS = 4096
LANES = 16
NCHUNK = COLS // LANES

NUM_CORES = 2
NUM_SUBCORES = 16
NUM_WORKERS = NUM_CORES * NUM_SUBCORES

SC_ROWS = 1024
RPW = SC_ROWS // NUM_WORKERS
SC_ROW0 = ROWS - SC_ROWS

TC_ROWS = ROWS - SC_ROWS
ROW_BLOCK = 128
NUM_BLOCKS = TC_ROWS // ROW_BLOCK
NBUF = 4
CHUNK_LO = TC_ROWS // NUM_WORKERS


def _argmax_body(x_hbm, idx_ref, buf, sem):
    i = pl.program_id(0)
    slot = lax.rem(i, NBUF)

    @pl.when(i == 0)
    def _():
        for j in range(NBUF - 1):
            pltpu.make_async_copy(
                x_hbm.at[pl.ds(j * ROW_BLOCK, ROW_BLOCK), 0],
                buf.at[j],
                sem.at[j],
            ).start()

    @pl.when(i + NBUF - 1 < NUM_BLOCKS)
    def _():
        nxt = lax.rem(i + NBUF - 1, NBUF)
        pltpu.make_async_copy(
            x_hbm.at[pl.ds((i + NBUF - 1) * ROW_BLOCK, ROW_BLOCK), 0],
            buf.at[nxt],
            sem.at[nxt],
        ).start()

    pltpu.make_async_copy(
        x_hbm.at[pl.ds(i * ROW_BLOCK, ROW_BLOCK), 0], buf.at[slot], sem.at[slot]
    ).wait()

    x = buf[slot]
    m = jnp.max(x, axis=1, keepdims=True)
    col = lax.broadcasted_iota(jnp.int32, x.shape, 1)
    cand = jnp.where(x == m, col, COLS)
    idx_ref[0, 0, :] = jnp.min(cand, axis=1)


_argmax_call = pl.pallas_call(
    _argmax_body,
    grid=(NUM_BLOCKS,),
    in_specs=[pl.BlockSpec(memory_space=pltpu.MemorySpace.HBM)],
    out_specs=pl.BlockSpec((1, 1, ROW_BLOCK), lambda i: (i, 0, 0)),
    out_shape=jax.ShapeDtypeStruct((NUM_BLOCKS, 1, ROW_BLOCK), jnp.int32),
    scratch_shapes=[
        pltpu.VMEM((NBUF, ROW_BLOCK, COLS), jnp.float32),
        pltpu.SemaphoreType.DMA((NBUF,)),
    ],
)

_SC_MESH = plsc.VectorSubcoreMesh(
    core_axis_name="c", subcore_axis_name="s", num_cores=NUM_CORES
)


def _row_argmax(rb):
    def chunk_body(c, carry):
        m, mi, cidx = carry
        v = rb[pl.ds(c * LANES, LANES)]
        pred = v > m
        m = jnp.where(pred, v, m)
        mi = jnp.where(pred, cidx, mi)
        return m, mi, cidx + LANES

    m0 = jnp.full((LANES,), -jnp.inf, jnp.float32)
    i0 = jnp.zeros((LANES,), jnp.int32)
    c0 = lax.iota(jnp.int32, LANES)
    m, mi, _ = lax.fori_loop(0, NCHUNK, chunk_body, (m0, i0, c0), unroll=8)
    gm = m
    for s in (1, 2, 4, 8):
        gm = jnp.maximum(gm, _rotate(gm, s))
    cand = jnp.where(m == gm, mi, COLS)
    for s in (1, 2, 4, 8):
        cand = jnp.minimum(cand, _rotate(cand, s))
    return cand


def _rotate(x, s):
    perm = ((lax.iota(jnp.int32, LANES) + s) & (LANES - 1)).reshape(LANES, 1)
    dnums = lax.GatherDimensionNumbers(
        offset_dims=(), collapsed_slice_dims=(0,), start_index_map=(0,)
    )
    return lax.gather(
        x, perm, dnums, (1,), mode=lax.GatherScatterMode.PROMISE_IN_BOUNDS
    )


@functools.partial(
    pl.kernel,
    out_type=jax.ShapeDtypeStruct((SC_ROWS,), jnp.float32),
    scratch_types=[
        pltpu.VMEM((2, COLS), jnp.float32),
        pltpu.VMEM((RPW,), jnp.int32),
        pltpu.VMEM((RPW,), jnp.float32),
        pltpu.SemaphoreType.DMA((2,)),
        pltpu.SemaphoreType.DMA,
    ],
    mesh=_SC_MESH,
    cost_estimate=pl.CostEstimate(
        flops=4 * SC_ROWS * COLS,
        bytes_accessed=4 * SC_ROWS * COLS,
        transcendentals=0,
    ),
)
def _sc_argmax_lookup(x_hbm, fsp_hbm, out_hbm, row_buf, idx_v, out_v, sems, gsem):
    wid = lax.axis_index("s") * NUM_CORES + lax.axis_index("c")
    r0 = SC_ROW0 + wid * RPW

    pltpu.make_async_copy(x_hbm.at[r0, 0], row_buf.at[0], sems.at[0]).start()
    kiota = lax.iota(jnp.int32, LANES)

    def group_body(g, _):
        acc = jnp.zeros((LANES,), jnp.int32)
        base = g * LANES
        for k in range(LANES):
            rr = base + k
            slot = k % 2
            nslot = 1 - slot

            @pl.when(rr + 1 < RPW)
            def _():
                pltpu.make_async_copy(
                    x_hbm.at[r0 + rr + 1, 0], row_buf.at[nslot], sems.at[nslot]
                ).start()

            pltpu.make_async_copy(
                x_hbm.at[r0 + rr, 0], row_buf.at[slot], sems.at[slot]
            ).wait()
            giv = _row_argmax(row_buf.at[slot])
            acc = jnp.where(kiota == k, giv, acc)
        idx_v[pl.ds(base, LANES)] = acc
        return 0

    lax.fori_loop(0, RPW // LANES, group_body, 0)
    pltpu.async_copy(fsp_hbm.at[idx_v], out_v, gsem).wait()
    pltpu.sync_copy(out_v, out_hbm.at[pl.ds(wid * RPW, RPW)])


@functools.partial(
    pl.kernel,
    out_type=jax.ShapeDtypeStruct((TC_ROWS,), jnp.float32),
    scratch_types=[
        pltpu.VMEM((CHUNK_LO,), jnp.int32),
        pltpu.VMEM((CHUNK_LO,), jnp.float32),
        pltpu.SemaphoreType.DMA,
    ],
    mesh=_SC_MESH,
)
def _sc_gather(idx_hbm, fsp_hbm, out_hbm, idx_v, out_v, sem):
    wid = lax.axis_index("s") * NUM_CORES + lax.axis_index("c")
    base = wid * CHUNK_LO
    pltpu.sync_copy(idx_hbm.at[pl.ds(base, CHUNK_LO)], idx_v)
    pltpu.async_copy(fsp_hbm.at[idx_v], out_v, sem).wait()
    pltpu.sync_copy(out_v, out_hbm.at[pl.ds(base, CHUNK_LO)])


def kernel(input, fspace):
    out_hi = _sc_argmax_lookup(input, fspace)
    idx_lo = _argmax_call(input).reshape(TC_ROWS)
    out_lo = _sc_gather(idx_lo, fspace)
    return jnp.concatenate([out_lo, out_hi])


# speedup vs baseline: 1.3754x; 1.0354x over previous
import functools

import jax
import jax.numpy as jnp
from jax import lax
from jax.experimental import pallas as pl
from jax.experimental.pallas import tpu as pltpu
from jax.experimental.pallas import tpu_sc as plsc

ROWS = 4096
COLS = 4096
LANES = 16
NCHUNK = COLS // LANES

NUM_CORES = 2
NUM_SUBCORES = 16
NUM_WORKERS = NUM_CORES * NUM_SUBCORES

SC_ROWS = 1024
RPW = SC_ROWS // NUM_WORKERS
SC_ROW0 = ROWS - SC_ROWS

TC_ROWS = ROWS - SC_ROWS
ROW_BLOCK = 128
NUM_BLOCKS = TC_ROWS // ROW_BLOCK
NBUF = 4
CHUNK_LO = TC_ROWS // NUM_WORKERS


def _argmax_body(x_hbm, fsp_ref, val_ref, buf, sem):
    i = pl.program_id(0)
    slot = lax.rem(i, NBUF)

    @pl.when(i == 0)
    def _():
        for j in range(NBUF - 1):
            pltpu.make_async_copy(
                x_hbm.at[pl.ds(j * ROW_BLOCK, ROW_BLOCK), 0],
                buf.at[j],
                sem.at[j],
            ).start()

    @pl.when(i + NBUF - 1 < NUM_BLOCKS)
    def _():
        nxt = lax.rem(i + NBUF - 1, NBUF)
        pltpu.make_async_copy(
            x_hbm.at[pl.ds((i + NBUF - 1) * ROW_BLOCK, ROW_BLOCK), 0],
            buf.at[nxt],
            sem.at[nxt],
        ).start()

    pltpu.make_async_copy(
        x_hbm.at[pl.ds(i * ROW_BLOCK, ROW_BLOCK), 0], buf.at[slot], sem.at[slot]
    ).wait()

    x = buf[slot]
    m = jnp.max(x, axis=1, keepdims=True)
    col = lax.broadcasted_iota(jnp.int32, x.shape, 1)
    cand = jnp.where(x == m, col, COLS)
    idx = jnp.min(cand, axis=1)
    onehot = col == idx[:, None]
    fsp = fsp_ref[0, :]
    val_ref[0, 0, :] = jnp.sum(jnp.where(onehot, fsp[None, :], 0.0), axis=1)


_argmax_call = pl.pallas_call(
    _argmax_body,
    grid=(NUM_BLOCKS,),
    in_specs=[
        pl.BlockSpec(memory_space=pltpu.MemorySpace.HBM),
        pl.BlockSpec((1, COLS), lambda i: (0, 0)),
    ],
    out_specs=pl.BlockSpec((1, 1, ROW_BLOCK), lambda i: (i, 0, 0)),
    out_shape=jax.ShapeDtypeStruct((NUM_BLOCKS, 1, ROW_BLOCK), jnp.float32),
    scratch_shapes=[
        pltpu.VMEM((NBUF, ROW_BLOCK, COLS), jnp.float32),
        pltpu.SemaphoreType.DMA((NBUF,)),
    ],
)

_SC_MESH = plsc.VectorSubcoreMesh(
    core_axis_name="c", subcore_axis_name="s", num_cores=NUM_CORES
)


NSEG = 4
SEG = COLS // NSEG


def _row_argmax(rb):
    def chunk_body(c, carry):
        ms, mis, cidx = carry
        base = c * LANES
        nms, nmis = [], []
        for p in range(NSEG):
            v = rb[pl.ds(p * SEG + base, LANES)]
            pred = v > ms[p]
            nms.append(jnp.where(pred, v, ms[p]))
            nmis.append(jnp.where(pred, cidx, mis[p]))
        return tuple(nms), tuple(nmis), cidx + LANES

    m0 = tuple(jnp.full((LANES,), -jnp.inf, jnp.float32) for _ in range(NSEG))
    i0 = tuple(jnp.zeros((LANES,), jnp.int32) for _ in range(NSEG))
    c0 = lax.iota(jnp.int32, LANES)
    ms, mis, _ = lax.fori_loop(
        0, SEG // LANES, chunk_body, (m0, i0, c0), unroll=8
    )
    m, mi = ms[0], mis[0]
    for p in range(1, NSEG):
        pred = ms[p] > m
        m = jnp.where(pred, ms[p], m)
        mi = jnp.where(pred, mis[p] + p * SEG, mi)
    gm = m
    for s in (1, 2, 4, 8):
        gm = jnp.maximum(gm, _rotate(gm, s))
    cand = jnp.where(m == gm, mi, COLS)
    for s in (1, 2, 4, 8):
        cand = jnp.minimum(cand, _rotate(cand, s))
    return cand


def _rotate(x, s):
    perm = ((lax.iota(jnp.int32, LANES) + s) & (LANES - 1)).reshape(LANES, 1)
    dnums = lax.GatherDimensionNumbers(
        offset_dims=(), collapsed_slice_dims=(0,), start_index_map=(0,)
    )
    return lax.gather(
        x, perm, dnums, (1,), mode=lax.GatherScatterMode.PROMISE_IN_BOUNDS
    )


@functools.partial(
    pl.kernel,
    out_type=jax.ShapeDtypeStruct((SC_ROWS,), jnp.float32),
    scratch_types=[
        pltpu.VMEM((2, COLS), jnp.float32),
        pltpu.VMEM((RPW,), jnp.int32),
        pltpu.VMEM((RPW,), jnp.float32),
        pltpu.SemaphoreType.DMA((2,)),
        pltpu.SemaphoreType.DMA,
    ],
    mesh=_SC_MESH,
    cost_estimate=pl.CostEstimate(
        flops=4 * SC_ROWS * COLS,
        bytes_accessed=4 * SC_ROWS * COLS,
        transcendentals=0,
    ),
)
def _sc_argmax_lookup(x_hbm, fsp_hbm, out_hbm, row_buf, idx_v, out_v, sems, gsem):
    wid = lax.axis_index("s") * NUM_CORES + lax.axis_index("c")
    r0 = SC_ROW0 + wid * RPW

    pltpu.make_async_copy(x_hbm.at[r0, 0], row_buf.at[0], sems.at[0]).start()
    kiota = lax.iota(jnp.int32, LANES)

    def group_body(g, _):
        acc = jnp.zeros((LANES,), jnp.int32)
        base = g * LANES
        for k in range(LANES):
            rr = base + k
            slot = k % 2
            nslot = 1 - slot

            @pl.when(rr + 1 < RPW)
            def _():
                pltpu.make_async_copy(
                    x_hbm.at[r0 + rr + 1, 0], row_buf.at[nslot], sems.at[nslot]
                ).start()

            pltpu.make_async_copy(
                x_hbm.at[r0 + rr, 0], row_buf.at[slot], sems.at[slot]
            ).wait()
            giv = _row_argmax(row_buf.at[slot])
            acc = jnp.where(kiota == k, giv, acc)
        idx_v[pl.ds(base, LANES)] = acc
        return 0

    lax.fori_loop(0, RPW // LANES, group_body, 0)
    pltpu.async_copy(fsp_hbm.at[idx_v], out_v, gsem).wait()
    pltpu.sync_copy(out_v, out_hbm.at[pl.ds(wid * RPW, RPW)])


def kernel(input, fspace):
    out_hi = _sc_argmax_lookup(input, fspace)
    out_lo = _argmax_call(input, fspace.reshape(1, COLS)).reshape(TC_ROWS)
    return jnp.concatenate([out_lo, out_hi])


# speedup vs baseline: 1.6417x; 1.1936x over previous
import functools

import jax
import jax.numpy as jnp
from jax import lax
from jax.experimental import pallas as pl
from jax.experimental.pallas import tpu as pltpu
from jax.experimental.pallas import tpu_sc as plsc

ROWS = 4096
COLS = 4096
LANES = 16
NCHUNK = COLS // LANES

NUM_CORES = 2
NUM_SUBCORES = 16
NUM_WORKERS = NUM_CORES * NUM_SUBCORES

SC_ROWS = 1024
RPW = SC_ROWS // NUM_WORKERS
GROW = 8
SC_ROW0 = ROWS - SC_ROWS

TC_ROWS = ROWS - SC_ROWS
ROW_BLOCK = 128
NUM_BLOCKS = TC_ROWS // ROW_BLOCK
NBUF = 4
CHUNK_LO = TC_ROWS // NUM_WORKERS


def _argmax_body(x_hbm, fsp_ref, val_ref, buf, sem):
    i = pl.program_id(0)
    slot = lax.rem(i, NBUF)

    @pl.when(i == 0)
    def _():
        for j in range(NBUF - 1):
            pltpu.make_async_copy(
                x_hbm.at[pl.ds(j * ROW_BLOCK, ROW_BLOCK), 0],
                buf.at[j],
                sem.at[j],
            ).start()

    @pl.when(i + NBUF - 1 < NUM_BLOCKS)
    def _():
        nxt = lax.rem(i + NBUF - 1, NBUF)
        pltpu.make_async_copy(
            x_hbm.at[pl.ds((i + NBUF - 1) * ROW_BLOCK, ROW_BLOCK), 0],
            buf.at[nxt],
            sem.at[nxt],
        ).start()

    pltpu.make_async_copy(
        x_hbm.at[pl.ds(i * ROW_BLOCK, ROW_BLOCK), 0], buf.at[slot], sem.at[slot]
    ).wait()

    x = buf[slot]
    m = jnp.max(x, axis=1, keepdims=True)
    col = lax.broadcasted_iota(jnp.int32, x.shape, 1)
    cand = jnp.where(x == m, col, COLS)
    idx = jnp.min(cand, axis=1)
    onehot = col == idx[:, None]
    fsp = fsp_ref[0, :]
    val_ref[0, 0, :] = jnp.sum(jnp.where(onehot, fsp[None, :], 0.0), axis=1)


_argmax_call = pl.pallas_call(
    _argmax_body,
    grid=(NUM_BLOCKS,),
    in_specs=[
        pl.BlockSpec(memory_space=pltpu.MemorySpace.HBM),
        pl.BlockSpec((1, COLS), lambda i: (0, 0)),
    ],
    out_specs=pl.BlockSpec((1, 1, ROW_BLOCK), lambda i: (i, 0, 0)),
    out_shape=jax.ShapeDtypeStruct((NUM_BLOCKS, 1, ROW_BLOCK), jnp.float32),
    scratch_shapes=[
        pltpu.VMEM((NBUF, ROW_BLOCK, COLS), jnp.float32),
        pltpu.SemaphoreType.DMA((NBUF,)),
    ],
)

_SC_MESH = plsc.VectorSubcoreMesh(
    core_axis_name="c", subcore_axis_name="s", num_cores=NUM_CORES
)


NSEG = 4
SEG = COLS // NSEG


def _row_argmax(rb):
    def chunk_body(c, carry):
        ms, mis, cidx = carry
        base = c * LANES
        nms, nmis = [], []
        for p in range(NSEG):
            v = rb[pl.ds(p * SEG + base, LANES)]
            pred = v > ms[p]
            nms.append(jnp.where(pred, v, ms[p]))
            nmis.append(jnp.where(pred, cidx, mis[p]))
        return tuple(nms), tuple(nmis), cidx + LANES

    m0 = tuple(jnp.full((LANES,), -jnp.inf, jnp.float32) for _ in range(NSEG))
    i0 = tuple(jnp.zeros((LANES,), jnp.int32) for _ in range(NSEG))
    c0 = lax.iota(jnp.int32, LANES)
    ms, mis, _ = lax.fori_loop(
        0, SEG // LANES, chunk_body, (m0, i0, c0), unroll=8
    )
    m, mi = ms[0], mis[0]
    for p in range(1, NSEG):
        pred = ms[p] > m
        m = jnp.where(pred, ms[p], m)
        mi = jnp.where(pred, mis[p] + p * SEG, mi)
    gm = m
    for s in (1, 2, 4, 8):
        gm = jnp.maximum(gm, _rotate(gm, s))
    cand = jnp.where(m == gm, mi, COLS)
    for s in (1, 2, 4, 8):
        cand = jnp.minimum(cand, _rotate(cand, s))
    return cand


def _rotate(x, s):
    perm = ((lax.iota(jnp.int32, LANES) + s) & (LANES - 1)).reshape(LANES, 1)
    dnums = lax.GatherDimensionNumbers(
        offset_dims=(), collapsed_slice_dims=(0,), start_index_map=(0,)
    )
    return lax.gather(
        x, perm, dnums, (1,), mode=lax.GatherScatterMode.PROMISE_IN_BOUNDS
    )


@functools.partial(
    pl.kernel,
    out_type=jax.ShapeDtypeStruct((SC_ROWS,), jnp.float32),
    scratch_types=[
        pltpu.VMEM((2, GROW, COLS), jnp.float32),
        pltpu.VMEM((RPW,), jnp.int32),
        pltpu.VMEM((RPW,), jnp.float32),
        pltpu.SemaphoreType.DMA((2,)),
        pltpu.SemaphoreType.DMA,
    ],
    mesh=_SC_MESH,
    cost_estimate=pl.CostEstimate(
        flops=4 * SC_ROWS * COLS,
        bytes_accessed=4 * SC_ROWS * COLS,
        transcendentals=0,
    ),
)
def _sc_argmax_lookup(x_hbm, fsp_hbm, out_hbm, row_buf, idx_v, out_v, sems, gsem):
    wid = lax.axis_index("s") * NUM_CORES + lax.axis_index("c")
    r0 = SC_ROW0 + wid * RPW
    ngroups = RPW // GROW

    pltpu.make_async_copy(
        x_hbm.at[pl.ds(r0, GROW), 0], row_buf.at[0], sems.at[0]
    ).start()
    kiota = lax.iota(jnp.int32, LANES)

    def group_body(g, acc):
        slot = lax.rem(g, 2)
        nslot = lax.rem(g + 1, 2)

        @pl.when(g + 1 < ngroups)
        def _():
            pltpu.make_async_copy(
                x_hbm.at[pl.ds(r0 + (g + 1) * GROW, GROW), 0],
                row_buf.at[nslot],
                sems.at[nslot],
            ).start()

        pltpu.make_async_copy(
            x_hbm.at[pl.ds(r0 + g * GROW, GROW), 0],
            row_buf.at[slot],
            sems.at[slot],
        ).wait()

        for k in range(GROW):
            giv = _row_argmax(row_buf.at[slot].at[k])
            acc = jnp.where(kiota == ((g * GROW + k) & (LANES - 1)), giv, acc)

        @pl.when(lax.rem(g, 2) == 1)
        def _():
            idx_v[pl.ds((g - 1) * GROW, LANES)] = acc

        return acc

    lax.fori_loop(0, ngroups, group_body, jnp.zeros((LANES,), jnp.int32))
    pltpu.async_copy(fsp_hbm.at[idx_v], out_v, gsem).wait()
    pltpu.sync_copy(out_v, out_hbm.at[pl.ds(wid * RPW, RPW)])


def kernel(input, fspace):
    out_hi = _sc_argmax_lookup(input, fspace)
    out_lo = _argmax_call(input, fspace.reshape(1, COLS)).reshape(TC_ROWS)
    return jnp.concatenate([out_lo, out_hi])


# speedup vs baseline: 1.6615x; 1.0120x over previous
import functools

import jax
import jax.numpy as jnp
from jax import lax
from jax.experimental import pallas as pl
from jax.experimental.pallas import tpu as pltpu
from jax.experimental.pallas import tpu_sc as plsc

ROWS = 4096
COLS = 4096
LANES = 16
NCHUNK = COLS // LANES

NUM_CORES = 2
NUM_SUBCORES = 16
NUM_WORKERS = NUM_CORES * NUM_SUBCORES

SC_ROWS = 1024
RPW = SC_ROWS // NUM_WORKERS
GROW = 8
SC_ROW0 = ROWS - SC_ROWS

TC_ROWS = ROWS - SC_ROWS
ROW_BLOCK = 128
NUM_BLOCKS = TC_ROWS // ROW_BLOCK
NBUF = 4
CHUNK_LO = TC_ROWS // NUM_WORKERS


def _argmax_body(x_hbm, fsp_ref, val_ref, buf, sem):
    i = pl.program_id(0)
    slot = lax.rem(i, NBUF)

    @pl.when(i == 0)
    def _():
        for j in range(NBUF - 1):
            pltpu.make_async_copy(
                x_hbm.at[pl.ds(j * ROW_BLOCK, ROW_BLOCK), 0],
                buf.at[j],
                sem.at[j],
            ).start()

    @pl.when(i + NBUF - 1 < NUM_BLOCKS)
    def _():
        nxt = lax.rem(i + NBUF - 1, NBUF)
        pltpu.make_async_copy(
            x_hbm.at[pl.ds((i + NBUF - 1) * ROW_BLOCK, ROW_BLOCK), 0],
            buf.at[nxt],
            sem.at[nxt],
        ).start()

    pltpu.make_async_copy(
        x_hbm.at[pl.ds(i * ROW_BLOCK, ROW_BLOCK), 0], buf.at[slot], sem.at[slot]
    ).wait()

    x = buf[slot]
    m = jnp.max(x, axis=1, keepdims=True)
    col = lax.broadcasted_iota(jnp.int32, x.shape, 1)
    cand = jnp.where(x == m, col, COLS)
    idx = jnp.min(cand, axis=1)
    onehot = col == idx[:, None]
    fsp = fsp_ref[0, :]
    val_ref[0, 0, :] = jnp.sum(jnp.where(onehot, fsp[None, :], 0.0), axis=1)


_argmax_call = pl.pallas_call(
    _argmax_body,
    grid=(NUM_BLOCKS,),
    in_specs=[
        pl.BlockSpec(memory_space=pltpu.MemorySpace.HBM),
        pl.BlockSpec((1, COLS), lambda i: (0, 0)),
    ],
    out_specs=pl.BlockSpec((1, 1, ROW_BLOCK), lambda i: (i, 0, 0)),
    out_shape=jax.ShapeDtypeStruct((NUM_BLOCKS, 1, ROW_BLOCK), jnp.float32),
    scratch_shapes=[
        pltpu.VMEM((NBUF, ROW_BLOCK, COLS), jnp.float32),
        pltpu.SemaphoreType.DMA((NBUF,)),
    ],
)

_SC_MESH = plsc.VectorSubcoreMesh(
    core_axis_name="c", subcore_axis_name="s", num_cores=NUM_CORES
)


NSEG = 4
SEG = COLS // NSEG


def _row_argmax(rb, perms):
    def chunk_body(c, carry):
        ms, mis, cidx = carry
        base = c * LANES
        nms, nmis = [], []
        for p in range(NSEG):
            v = rb[pl.ds(p * SEG + base, LANES)]
            pred = v > ms[p]
            nms.append(jnp.where(pred, v, ms[p]))
            nmis.append(jnp.where(pred, cidx, mis[p]))
        return tuple(nms), tuple(nmis), cidx + LANES

    m0 = tuple(jnp.full((LANES,), -jnp.inf, jnp.float32) for _ in range(NSEG))
    i0 = tuple(jnp.zeros((LANES,), jnp.int32) for _ in range(NSEG))
    c0 = lax.iota(jnp.int32, LANES)
    ms, mis, _ = lax.fori_loop(
        0, SEG // LANES, chunk_body, (m0, i0, c0), unroll=4
    )
    m, mi = ms[0], mis[0]
    for p in range(1, NSEG):
        pred = ms[p] > m
        m = jnp.where(pred, ms[p], m)
        mi = jnp.where(pred, mis[p] + p * SEG, mi)
    gm = m
    for perm in perms:
        gm = jnp.maximum(gm, _permute(gm, perm))
    cand = jnp.where(m == gm, mi, COLS)
    for perm in perms:
        cand = jnp.minimum(cand, _permute(cand, perm))
    return cand


def _permute(x, perm):
    dnums = lax.GatherDimensionNumbers(
        offset_dims=(), collapsed_slice_dims=(0,), start_index_map=(0,)
    )
    return lax.gather(
        x, perm, dnums, (1,), mode=lax.GatherScatterMode.PROMISE_IN_BOUNDS
    )


def _butterfly_perms():
    ii = lax.iota(jnp.int32, LANES)
    return [((ii + s) & (LANES - 1)).reshape(LANES, 1) for s in (1, 2, 4, 8)]


@functools.partial(
    pl.kernel,
    out_type=jax.ShapeDtypeStruct((SC_ROWS,), jnp.float32),
    scratch_types=[
        pltpu.VMEM((2, GROW, COLS), jnp.float32),
        pltpu.VMEM((RPW,), jnp.int32),
        pltpu.VMEM((RPW,), jnp.float32),
        pltpu.SemaphoreType.DMA((2,)),
        pltpu.SemaphoreType.DMA,
    ],
    mesh=_SC_MESH,
    cost_estimate=pl.CostEstimate(
        flops=4 * SC_ROWS * COLS,
        bytes_accessed=4 * SC_ROWS * COLS,
        transcendentals=0,
    ),
)
def _sc_argmax_lookup(x_hbm, fsp_hbm, out_hbm, row_buf, idx_v, out_v, sems, gsem):
    wid = lax.axis_index("s") * NUM_CORES + lax.axis_index("c")
    r0 = SC_ROW0 + wid * RPW
    ngroups = RPW // GROW

    pltpu.make_async_copy(
        x_hbm.at[pl.ds(r0, GROW), 0], row_buf.at[0], sems.at[0]
    ).start()
    kiota = lax.iota(jnp.int32, LANES)
    perms = _butterfly_perms()

    def group_body(g, acc):
        slot = lax.rem(g, 2)
        nslot = lax.rem(g + 1, 2)

        @pl.when(g + 1 < ngroups)
        def _():
            pltpu.make_async_copy(
                x_hbm.at[pl.ds(r0 + (g + 1) * GROW, GROW), 0],
                row_buf.at[nslot],
                sems.at[nslot],
            ).start()

        pltpu.make_async_copy(
            x_hbm.at[pl.ds(r0 + g * GROW, GROW), 0],
            row_buf.at[slot],
            sems.at[slot],
        ).wait()

        def row_body(k, acc):
            giv = _row_argmax(row_buf.at[slot].at[k], perms)
            return jnp.where(kiota == ((g * GROW + k) & (LANES - 1)), giv, acc)

        acc = lax.fori_loop(0, GROW, row_body, acc)

        @pl.when(lax.rem(g, 2) == 1)
        def _():
            idx_v[pl.ds((g - 1) * GROW, LANES)] = acc

        return acc

    lax.fori_loop(0, ngroups, group_body, jnp.zeros((LANES,), jnp.int32))
    pltpu.async_copy(fsp_hbm.at[idx_v], out_v, gsem).wait()
    pltpu.sync_copy(out_v, out_hbm.at[pl.ds(wid * RPW, RPW)])


def kernel(input, fspace):
    out_hi = _sc_argmax_lookup(input, fspace)
    out_lo = _argmax_call(input, fspace.reshape(1, COLS)).reshape(TC_ROWS)
    return jnp.concatenate([out_lo, out_hi])
